# baseline (speedup 1.0000x reference)
import jax
import jax.numpy as jnp
from jax import lax
from jax.experimental import pallas as pl
from jax.experimental.pallas import tpu as pltpu
from jax.experimental.pallas import tpu_sc as plsc

N = 524288
WIDTH = 128
NUM_IMAGES = 1024
K = NUM_IMAGES * 32
M_BUF = 33024
SENTINEL_ID = NUM_IMAGES


def _iota16():
    return lax.broadcasted_iota(jnp.int32, (16,), 0)


def _sc_mesh(num_cores):
    return plsc.VectorSubcoreMesh(
        core_axis_name="c", subcore_axis_name="s", num_cores=num_cores)


_SC_PARAMS = pltpu.CompilerParams(needs_layout_passes=False)


_SUB_A = 2048


def _bincount_body(ids_hbm, p_hbm, buf, tail16, s_loc, e_loc,
                   sh_s, sh_e, tmp64, acc_s, acc_e):
    wid = lax.axis_index("s")
    zeros16 = jnp.zeros((16,), jnp.int32)
    iota = _iota16()
    base = wid * _BCH

    def zrow(i, _):
        s_loc[pl.ds(i * 16, 16)] = zeros16
        e_loc[pl.ds(i * 16, 16)] = zeros16
        return 0
    lax.fori_loop(0, NUM_IMAGES // 16, zrow, 0)

    pltpu.sync_copy(ids_hbm.at[pl.ds(base, _BCH)], buf.at[pl.ds(8, _BCH)])

    @pl.when(wid > 0)
    def _():
        pltpu.sync_copy(ids_hbm.at[pl.ds(base - 8, 8)], buf.at[pl.ds(0, 8)])

    @pl.when(wid < 15)
    def _():
        pltpu.sync_copy(ids_hbm.at[pl.ds(base + _BCH, 8)],
                        buf.at[pl.ds(8 + _BCH, 8)])

    def vec(i, _):
        for k in range(4):
            o = i * 64 + k * 16
            vcur = buf[pl.ds(8 + o, 16)]
            vprev = buf[pl.ds(7 + o, 16)]
            jvec = base + o + iota
            first = (wid == 0) & (i == 0) & (k == 0)
            fvec = (iota == 0) & first
            change = vcur != vprev
            mstart = change | fvec
            mend = change & jnp.logical_not(fvec)
            plsc.store_scatter(s_loc, [vcur], jvec, mask=mstart)
            plsc.store_scatter(e_loc, [vprev], jvec, mask=mend)
        return 0
    lax.fori_loop(0, _BCH // 64, vec, 0)

    @pl.when(wid == 15)
    def _():
        pltpu.sync_copy(ids_hbm.at[pl.ds(N - 16, 16)], tail16)
        v = tail16[...]
        plsc.store_scatter(e_loc, [v],
                           jnp.full((16,), N, jnp.int32), mask=(iota == 15))

    pltpu.sync_copy(s_loc, sh_s.at[wid])
    pltpu.sync_copy(e_loc, sh_e.at[wid])
    plsc.subcore_barrier()

    col = wid * 64
    for c in range(4):
        acc_s[pl.ds(c * 16, 16)] = zeros16
        acc_e[pl.ds(c * 16, 16)] = zeros16
    for r in range(16):
        pltpu.sync_copy(sh_s.at[r, pl.ds(col, 64)], tmp64)
        for c in range(4):
            sl = pl.ds(c * 16, 16)
            acc_s[sl] = acc_s[sl] + tmp64[sl]
        pltpu.sync_copy(sh_e.at[r, pl.ds(col, 64)], tmp64)
        for c in range(4):
            sl = pl.ds(c * 16, 16)
            acc_e[sl] = acc_e[sl] + tmp64[sl]
    for c in range(4):
        sl = pl.ds(c * 16, 16)
        acc_s[sl] = acc_e[sl] - acc_s[sl]
    pltpu.sync_copy(acc_s, p_hbm.at[pl.ds(col, 64)])


def _bincount(image_id):
    return pl.kernel(
        _bincount_body,
        out_type=jax.ShapeDtypeStruct((NUM_IMAGES,), jnp.int32),
        mesh=_sc_mesh(1),
        compiler_params=_SC_PARAMS,
        scratch_types=[
            pltpu.VMEM((_BCH + 16,), jnp.int32),
            pltpu.VMEM((16,), jnp.int32),
            pltpu.VMEM((NUM_IMAGES,), jnp.int32),
            pltpu.VMEM((NUM_IMAGES,), jnp.int32),
            pltpu.VMEM_SHARED((16, NUM_IMAGES), jnp.int32),
            pltpu.VMEM_SHARED((16, NUM_IMAGES), jnp.int32),
            pltpu.VMEM((64,), jnp.int32),
            pltpu.VMEM((64,), jnp.int32),
            pltpu.VMEM((64,), jnp.int32),
        ],
    )(image_id)


_BCH = N // 16


def _radix_body(g_hbm, ids_hbm, ltab_hbm, meta_hbm, cnts_hbm,
                gbuf, idbuf, ltab, ubuf, hist_loc, hsum, tmp4k, hscan,
                sc16a, sc16b, crow, hist_sh):
    wid = lax.axis_index("s")
    iota = _iota16()
    ones16 = jnp.ones((16,), jnp.int32)
    zeros16 = jnp.zeros((16,), jnp.int32)
    lane_off = iota * 2048

    def u32vec(x):
        return lax.broadcast(x, (16,)).astype(jnp.uint32)

    sc16a[pl.ds(16, 16)] = zeros16
    sc16b[pl.ds(16, 16)] = zeros16

    pltpu.sync_copy(ltab_hbm, ltab)
    half = _BCH // 2
    for h in range(2):
        pltpu.sync_copy(g_hbm.at[pl.ds(wid * _BCH + h * half, half)], gbuf)
        pltpu.sync_copy(ids_hbm.at[pl.ds(wid * _BCH + h * half, half)],
                        idbuf)

        def prep(i, _):
            for k in range(4):
                o = i * 64 + k * 16
                lt = plsc.load_gather(ltab, [idbuf[pl.ds(o, 16)]])
                s = lt + gbuf[pl.ds(o, 16)]
                bb = plsc.bitcast(s, jnp.uint32)
                hi = bb >= jnp.uint32(0x80000000)
                ubuf[pl.ds(h * half + o, 16)] = jnp.where(
                    hi, ~bb, bb | jnp.uint32(0x80000000))
            return 0
        lax.fori_loop(0, half // 64, prep, 0)

    def do_pass(nbins, bin_of, mask_of, target):
        nrows = nbins // 16

        def z(i, _):
            for k in range(8):
                hist_loc[pl.ds(i * 128 + k * 16, 16)] = zeros16
            return 0
        lax.fori_loop(0, 16 * 2048 // 128, z, 0)

        def vec(i, _):
            for k in range(4):
                u = ubuf[pl.ds(i * 64 + k * 16, 16)]
                idx = lane_off + bin_of(u)
                plsc.addupdate_scatter(hist_loc, [idx], ones16,
                                       mask=mask_of(u))
            return 0
        lax.fori_loop(0, _BCH // 64, vec, 0)

        def lr(c, _):
            acc = hist_loc[pl.ds(c * 16, 16)]
            for rr in range(1, 16):
                acc = acc + hist_loc[pl.ds(rr * 2048 + c * 16, 16)]
            hsum[pl.ds(c * 16, 16)] = acc
            return 0
        lax.fori_loop(0, nrows, lr, 0)
        pltpu.sync_copy(hsum.at[pl.ds(0, nbins)],
                        hist_sh.at[wid, pl.ds(0, nbins)])
        plsc.subcore_barrier()

        def zh(c, _):
            for k in range(4):
                hscan[pl.ds(c * 64 + k * 16, 16)] = zeros16
            return 0
        lax.fori_loop(0, nrows // 4, zh, 0)
        for rr in range(16):
            pltpu.sync_copy(hist_sh.at[rr, pl.ds(0, nbins)],
                            tmp4k.at[pl.ds(0, nbins)])

            def addrow(c, _):
                for k in range(4):
                    sl = pl.ds(c * 64 + k * 16, 16)
                    hscan[sl] = hscan[sl] + tmp4k[sl]
                return 0
            lax.fori_loop(0, nrows // 4, addrow, 0)
        plsc.subcore_barrier()

        def step(r2, carry):
            acc, found, b, above = carry
            r = nrows - 1 - r2
            row = hscan[pl.ds(r * 16, 16)]
            total = jnp.sum(row)
            crossed = jnp.logical_and(found == 0, acc + total >= target)

            def on_cross(_):
                rrev = lax.rev(row, (0,))
                suf = lax.rev(plsc.cumsum(rrev), (0,))
                cmask = ((acc + suf) >= target).astype(jnp.int32)
                ffs = plsc.all_reduce_ffs(lax.rev(cmask, (0,)) != 0)
                f0 = ffs[0] if getattr(ffs, "ndim", 0) else ffs
                lstar = 15 - f0
                sc16a[pl.ds(0, 16)] = suf
                sc16b[pl.ds(0, 16)] = row
                sufl = sc16a[pl.ds(lstar, 16)][0]
                rowl = sc16b[pl.ds(lstar, 16)][0]
                return r * 16 + lstar, acc + sufl - rowl

            def no_cross(_):
                return b, above

            b_n, ab_n = lax.cond(crossed, on_cross, no_cross, 0)
            return (acc + total, found | crossed.astype(jnp.int32),
                    b_n, ab_n)

        _, _, b, above = lax.fori_loop(
            0, nrows, step,
            (jnp.int32(0), jnp.int32(0), jnp.int32(0), jnp.int32(0)))
        return b, above

    b1, ab1 = do_pass(
        2048,
        lambda u: (u >> 21).astype(jnp.int32),
        lambda u: None,
        jnp.int32(K))
    k2 = K - ab1
    b2, ab2 = do_pass(
        2048,
        lambda u: ((u >> 10) & jnp.uint32(0x7FF)).astype(jnp.int32),
        lambda u: (u >> 21) == u32vec(b1),
        k2)
    k3 = k2 - ab2
    b3, ab3 = do_pass(
        1024,
        lambda u: (u & jnp.uint32(0x3FF)).astype(jnp.int32),
        lambda u: (u >> 10) == u32vec((b1 << 11) | b2),
        k3)
    r = k3 - ab3
    t_i32 = (b1 << 21) | (b2 << 10) | b3
    tvec = plsc.bitcast(lax.broadcast(t_i32, (16,)), jnp.uint32)

    def count_half(h):
        def vec(i, c2):
            g, ti = c2
            u = ubuf[pl.ds(h * (_BCH // 2) + i * 16, 16)]
            g = g + plsc.all_reduce_population_count(u > tvec)
            ti = ti + plsc.all_reduce_population_count(u == tvec)
            return g, ti
        return lax.fori_loop(0, _BCH // 2 // 16, vec, (zeros16, zeros16))

    g0, t0 = count_half(0)
    g1, t1 = count_half(1)
    zf = zeros16
    vout = (jnp.where(iota == 0, g0, zf) + jnp.where(iota == 1, t0, zf)
            + jnp.where(iota == 8, g1, zf) + jnp.where(iota == 9, t1, zf))
    crow[pl.ds(0, 16)] = vout
    pltpu.sync_copy(crow, cnts_hbm.at[pl.ds(wid * 16, 16)])

    @pl.when(wid == 0)
    def _():
        tv = lax.broadcast(t_i32, (16,))
        rv = lax.broadcast(r, (16,))
        mv = jnp.where(iota == 0, tv, jnp.where(iota == 1, rv, zf))
        crow[pl.ds(0, 16)] = mv
        pltpu.sync_copy(crow, meta_hbm)


def _threshold(g, image_id, logtab):
    return pl.kernel(
        _radix_body,
        out_type=(jax.ShapeDtypeStruct((16,), jnp.int32),
                  jax.ShapeDtypeStruct((_NW * 8,), jnp.int32)),
        mesh=_sc_mesh(1),
        compiler_params=_SC_PARAMS,
        scratch_types=[
            pltpu.VMEM((_BCH // 2,), jnp.float32),
            pltpu.VMEM((_BCH // 2,), jnp.int32),
            pltpu.VMEM((NUM_IMAGES,), jnp.float32),
            pltpu.VMEM((_BCH,), jnp.uint32),
            pltpu.VMEM((16 * 2048,), jnp.int32),
            pltpu.VMEM((2048,), jnp.int32),
            pltpu.VMEM((2048,), jnp.int32),
            pltpu.VMEM((2048,), jnp.int32),
            pltpu.VMEM((32,), jnp.int32),
            pltpu.VMEM((32,), jnp.int32),
            pltpu.VMEM((16,), jnp.int32),
            pltpu.VMEM_SHARED((16, 2048), jnp.int32),
        ],
    )(g, image_id, logtab)


_NW = 32
_CHUNK_C = N // _NW
_SUB_C = 2048
_LOC = 16640


def _compact_body(g_hbm, ids_hbm, ltab_hbm, data_hbm, meta_hbm, cnts_hbm,
                  x_hbm, selid_hbm,
                  mbuf, cbuf, sbuf, ibuf, ltab, idxloc, idloc,
                  rows128, rows8x, rows8, sem_g0, sem_g1):
    wid = lax.axis_index("s") * 2 + lax.axis_index("c")
    iota = _iota16()

    pltpu.sync_copy(meta_hbm, mbuf)
    pltpu.sync_copy(cnts_hbm, cbuf.at[pl.ds(0, _NW * 8)])
    mvec = mbuf[pl.ds(0, 16)]
    t_i32 = mvec[0]
    r = mvec[1]
    tvec = plsc.bitcast(lax.broadcast(t_i32, (16,)), jnp.uint32)

    def pref(w2, carry):
        off, tiepre, my_n, my_a = carry
        cv = cbuf[pl.ds(w2 * 8, 16)]
        gt = cv[0]
        tie = cv[1]
        a = jnp.clip(r - tiepre, 0, tie)
        nw = gt + a
        mw = (nw + 7) & (-8)
        off = jnp.where(w2 < wid, off + mw, off)
        my_n = jnp.where(w2 == wid, nw, my_n)
        my_a = jnp.where(w2 == wid, a, my_a)
        tiepre = jnp.where(w2 < wid, tiepre + tie, tiepre)
        return off, tiepre, my_n, my_a
    off, _, n_w, a_w = lax.fori_loop(
        0, _NW, pref, (jnp.int32(0), jnp.int32(0), jnp.int32(0),
                       jnp.int32(0)))
    m_w = jnp.where(wid == _NW - 1, M_BUF - off, (n_w + 7) & (-8))
    off = pl.multiple_of(off, 8)
    m_w = pl.multiple_of(m_w, 8)

    chunk_base = wid * _CHUNK_C
    zero_splat = jnp.zeros((16,), jnp.int32)

    pltpu.sync_copy(g_hbm.at[pl.ds(chunk_base, _CHUNK_C)], sbuf)
    pltpu.sync_copy(ids_hbm.at[pl.ds(chunk_base, _CHUNK_C)], ibuf)
    pltpu.sync_copy(ltab_hbm, ltab)

    def monotone(o):
        lt = plsc.load_gather(ltab, [ibuf[pl.ds(o, 16)]])
        s = lt + sbuf[pl.ds(o, 16)]
        bb = plsc.bitcast(s, jnp.uint32)
        hi = bb >= jnp.uint32(0x80000000)
        return jnp.where(hi, ~bb, bb | jnp.uint32(0x80000000))

    def emit(o, pos, mask):
        idxvec = chunk_base + o + iota
        plsc.store_scatter(idxloc, [pos], idxvec, mask=mask)
        idv = ibuf[pl.ds(o, 16)]
        plsc.store_scatter(idloc, [pos], idv, mask=mask)

    def vec_ties(i, vc):
        ptr, tiec = vc
        for k in range(2):
            o = i * 32 + k * 16
            u = monotone(o)
            mgt = u > tvec
            mtie = u == tvec
            cumt = plsc.cumsum(mtie.astype(jnp.int32))
            rank = tiec + cumt - 1
            macc = mtie & (rank < a_w)
            mask = mgt | macc
            cums = plsc.cumsum(mask.astype(jnp.int32))
            pos = ptr + cums - 1
            emit(o, pos, mask)
            ptr = ptr + plsc.all_reduce_population_count(mask)
            tiec = tiec + plsc.all_reduce_population_count(mtie)
        return ptr, tiec

    def vec_plain(i, ptr):
        for k in range(2):
            o = i * 32 + k * 16
            u = monotone(o)
            mask = u > tvec
            cums = plsc.cumsum(mask.astype(jnp.int32))
            pos = ptr + cums - 1
            emit(o, pos, mask)
            ptr = ptr + plsc.all_reduce_population_count(mask)
        return ptr

    def scan_ties(_):
        return lax.fori_loop(0, _CHUNK_C // 32, vec_ties,
                             (zero_splat, zero_splat))[0]

    def scan_plain(_):
        return lax.fori_loop(0, _CHUNK_C // 32, vec_plain, zero_splat)

    lax.cond(a_w > 0, scan_ties, scan_plain, 0)

    sent = jnp.full((16,), SENTINEL_ID, jnp.int32)

    def fill(j, _):
        pos = n_w + j * 16 + iota
        mask = pos < m_w
        plsc.store_scatter(idxloc, [pos], zero_splat, mask=mask)
        plsc.store_scatter(idloc, [pos], sent, mask=mask)
        return 0
    lax.fori_loop(0, (m_w - n_w + 15) // 16, fill, 0)

    nfull = m_w // 128
    gbufs = [(rows128, sem_g0), (rows8x, sem_g1)]
    descs = [None, None]
    _MAXCH = _LOC // 128

    for c in range(_MAXCH + 1):
        if c < _MAXCH:
            buf_c, sem_c = gbufs[c % 2]
            lo_c = c * 128

            @pl.when(c < nfull)
            def _(buf_c=buf_c, sem_c=sem_c, lo_c=lo_c):
                pltpu.async_copy(
                    data_hbm.at[idxloc.at[pl.ds(lo_c, 128)]], buf_c, sem_c)
        if c >= 1:
            p = c - 1
            buf_p, sem_p = gbufs[p % 2]
            lo_p = p * 128

            @pl.when(p < nfull)
            def _(buf_p=buf_p, sem_p=sem_p, lo_p=lo_p):
                pltpu.make_async_copy(
                    data_hbm.at[idxloc.at[pl.ds(lo_p, 128)]], buf_p,
                    sem_p).wait()
                dst = pl.multiple_of(off + lo_p, 8)
                pltpu.sync_copy(buf_p, x_hbm.at[pl.ds(dst, 128)])
                pltpu.sync_copy(idloc.at[pl.ds(lo_p, 128)],
                                selid_hbm.at[pl.ds(dst, 128)])

    ntail = (m_w - nfull * 128) // 8

    def gtail(c, _):
        lo = pl.multiple_of(nfull * 128 + c * 8, 8)
        dst = pl.multiple_of(off + lo, 8)
        pltpu.sync_copy(data_hbm.at[idxloc.at[pl.ds(lo, 8)]], rows8)
        pltpu.sync_copy(rows8, x_hbm.at[pl.ds(dst, 8)])
        pltpu.sync_copy(idloc.at[pl.ds(lo, 8)],
                        selid_hbm.at[pl.ds(dst, 8)])
        return 0
    lax.fori_loop(0, ntail, gtail, 0)


def _compact_gather(data, image_id, g, logtab, meta, cnts):
    return pl.kernel(
        _compact_body,
        out_type=(jax.ShapeDtypeStruct((M_BUF, WIDTH), jnp.float32),
                  jax.ShapeDtypeStruct((M_BUF,), jnp.int32)),
        mesh=_sc_mesh(2),
        compiler_params=_SC_PARAMS,
        scratch_types=[
            pltpu.VMEM((16,), jnp.int32),
            pltpu.VMEM((_NW * 8 + 16,), jnp.int32),
            pltpu.VMEM((_CHUNK_C,), jnp.float32),
            pltpu.VMEM((_CHUNK_C,), jnp.int32),
            pltpu.VMEM((NUM_IMAGES,), jnp.float32),
            pltpu.VMEM((_LOC,), jnp.int32),
            pltpu.VMEM((_LOC,), jnp.int32),
            pltpu.VMEM((128, WIDTH), jnp.float32),
            pltpu.VMEM((128, WIDTH), jnp.float32),
            pltpu.VMEM((8, WIDTH), jnp.float32),
            pltpu.SemaphoreType.DMA,
            pltpu.SemaphoreType.DMA,
        ],
    )(g, image_id, logtab, data, meta, cnts)


def _mlp_kernel(x_ref, w0_ref, b0_ref, w1_ref, b1_ref, o_ref):
    h = jnp.dot(x_ref[...], w0_ref[...], preferred_element_type=jnp.float32)
    h = jnp.maximum(h + b0_ref[...], 0.0)
    o = jnp.dot(h, w1_ref[...], preferred_element_type=jnp.float32)
    o_ref[...] = o + b1_ref[...]


def _mlp(x, W0, b0, W1s, b1s):
    rows = 256
    grid = (M_BUF // rows,)
    return pl.pallas_call(
        _mlp_kernel,
        grid=grid,
        in_specs=[
            pl.BlockSpec((rows, WIDTH), lambda i: (i, 0)),
            pl.BlockSpec((WIDTH, WIDTH), lambda i: (0, 0)),
            pl.BlockSpec((1, WIDTH), lambda i: (0, 0)),
            pl.BlockSpec((WIDTH, WIDTH), lambda i: (0, 0)),
            pl.BlockSpec((1, WIDTH), lambda i: (0, 0)),
        ],
        out_specs=pl.BlockSpec((rows, WIDTH), lambda i: (i, 0)),
        out_shape=jax.ShapeDtypeStruct((M_BUF, WIDTH), jnp.float32),
    )(x, W0, b0.reshape(1, WIDTH), W1s, b1s.reshape(1, WIDTH))


_E_ROWS = M_BUF // 16
_E_FULL = _E_ROWS // 128
_E_TAIL = _E_ROWS - _E_FULL * 128
_ACC_ROWS = 1040


def _segsum_body(rows_hbm, ids_hbm, out_hbm,
                 zbuf, ids_a, rows_a, ids_b, rows_b, ids16, rows16, obuf,
                 sem_a, sem_b, acc_sh):
    wid = lax.axis_index("s")
    z16 = jnp.zeros((16,), jnp.float32)

    def zrow(r, _):
        for c in range(WIDTH // 16):
            zbuf[r, pl.ds(c * 16, 16)] = z16
        return 0
    lax.fori_loop(0, _ACC_ROWS // 16, zrow, 0)
    pltpu.sync_copy(zbuf, acc_sh.at[pl.ds(wid * (_ACC_ROWS // 16),
                                          _ACC_ROWS // 16)])
    plsc.subcore_barrier()

    base = wid * _E_ROWS
    bufs = [(ids_a, rows_a, sem_a), (ids_b, rows_b, sem_b)]

    def issue(c):
        ids_r, rows_r, sem = bufs[c % 2]
        off = base + c * 128
        d1 = pltpu.async_copy(ids_hbm.at[pl.ds(off, 128)], ids_r, sem)
        d2 = pltpu.async_copy(rows_hbm.at[pl.ds(off, 128)], rows_r, sem)
        return d1, d2

    pend = issue(0)
    for c in range(_E_FULL):
        pend[0].wait()
        pend[1].wait()
        if c + 1 < _E_FULL:
            nxt = issue(c + 1)
        ids_r, rows_r, _ = bufs[c % 2]
        pltpu.sync_copy(rows_r, acc_sh.at[ids_r], add=True)
        if c + 1 < _E_FULL:
            pend = nxt
    toff = base + _E_FULL * 128
    pltpu.sync_copy(ids_hbm.at[pl.ds(toff, _E_TAIL)], ids16)
    pltpu.sync_copy(rows_hbm.at[pl.ds(toff, _E_TAIL)], rows16)
    pltpu.sync_copy(rows16, acc_sh.at[ids16], add=True)
    plsc.subcore_barrier()

    pltpu.sync_copy(acc_sh.at[pl.ds(wid * 64, 64)], obuf)
    pltpu.sync_copy(obuf, out_hbm.at[pl.ds(wid * 64, 64)])


def _segsum(out_rows, sel_ids):
    return pl.kernel(
        _segsum_body,
        out_type=jax.ShapeDtypeStruct((NUM_IMAGES, WIDTH), jnp.float32),
        mesh=_sc_mesh(1),
        compiler_params=_SC_PARAMS,
        scratch_types=[
            pltpu.VMEM((_ACC_ROWS // 16, WIDTH), jnp.float32),
            pltpu.VMEM((128,), jnp.int32),
            pltpu.VMEM((128, WIDTH), jnp.float32),
            pltpu.VMEM((128,), jnp.int32),
            pltpu.VMEM((128, WIDTH), jnp.float32),
            pltpu.VMEM((16,), jnp.int32),
            pltpu.VMEM((16, WIDTH), jnp.float32),
            pltpu.VMEM((64, WIDTH), jnp.float32),
            pltpu.SemaphoreType.DMA,
            pltpu.SemaphoreType.DMA,
            pltpu.VMEM_SHARED((_ACC_ROWS, WIDTH), jnp.float32),
        ],
    )(out_rows, sel_ids)


def kernel(data, image_id, sample_size, W0, b0, W1, b1):
    counts = _bincount(image_id)
    logtab = jnp.log((jnp.float32(1.0) / counts.astype(jnp.float32))
                     / jnp.float32(NUM_IMAGES))
    g = jax.random.gumbel(jax.random.key(1234), (N,), dtype=jnp.float32)

    meta, cnts = _threshold(g, image_id, logtab)
    x, ids = _compact_gather(data, image_id, g, logtab, meta, cnts)

    inv = 1.0 / jnp.asarray(sample_size).astype(jnp.float32)
    out_rows = _mlp(x, W0, b0, W1 * inv, b1 * inv)
    return _segsum(out_rows, ids)


# speedup vs baseline: 1.0068x; 1.0068x over previous
import jax
import jax.numpy as jnp
from jax import lax
from jax.experimental import pallas as pl
from jax.experimental.pallas import tpu as pltpu
from jax.experimental.pallas import tpu_sc as plsc

N = 524288
WIDTH = 128
NUM_IMAGES = 1024
K = NUM_IMAGES * 32
M_BUF = 33024
SENTINEL_ID = NUM_IMAGES


def _iota16():
    return lax.broadcasted_iota(jnp.int32, (16,), 0)


def _sc_mesh(num_cores):
    return plsc.VectorSubcoreMesh(
        core_axis_name="c", subcore_axis_name="s", num_cores=num_cores)


_SC_PARAMS = pltpu.CompilerParams(needs_layout_passes=False)


_SUB_A = 2048


def _bincount_body(ids_hbm, p_hbm, buf, tail16, s_loc, e_loc,
                   sh_s, sh_e, tmp64, acc_s, acc_e):
    wid = lax.axis_index("s")
    zeros16 = jnp.zeros((16,), jnp.int32)
    iota = _iota16()
    base = wid * _BCH

    def zrow(i, _):
        s_loc[pl.ds(i * 16, 16)] = zeros16
        e_loc[pl.ds(i * 16, 16)] = zeros16
        return 0
    lax.fori_loop(0, NUM_IMAGES // 16, zrow, 0)

    pltpu.sync_copy(ids_hbm.at[pl.ds(base, _BCH)], buf.at[pl.ds(8, _BCH)])

    @pl.when(wid > 0)
    def _():
        pltpu.sync_copy(ids_hbm.at[pl.ds(base - 8, 8)], buf.at[pl.ds(0, 8)])

    @pl.when(wid < 15)
    def _():
        pltpu.sync_copy(ids_hbm.at[pl.ds(base + _BCH, 8)],
                        buf.at[pl.ds(8 + _BCH, 8)])

    def vec(i, _):
        for k in range(4):
            o = i * 64 + k * 16
            vcur = buf[pl.ds(8 + o, 16)]
            vprev = buf[pl.ds(7 + o, 16)]
            jvec = base + o + iota
            first = (wid == 0) & (i == 0) & (k == 0)
            fvec = (iota == 0) & first
            change = vcur != vprev
            mstart = change | fvec
            mend = change & jnp.logical_not(fvec)
            plsc.store_scatter(s_loc, [vcur], jvec, mask=mstart)
            plsc.store_scatter(e_loc, [vprev], jvec, mask=mend)
        return 0
    lax.fori_loop(0, _BCH // 64, vec, 0)

    @pl.when(wid == 15)
    def _():
        pltpu.sync_copy(ids_hbm.at[pl.ds(N - 16, 16)], tail16)
        v = tail16[...]
        plsc.store_scatter(e_loc, [v],
                           jnp.full((16,), N, jnp.int32), mask=(iota == 15))

    pltpu.sync_copy(s_loc, sh_s.at[wid])
    pltpu.sync_copy(e_loc, sh_e.at[wid])
    plsc.subcore_barrier()

    col = wid * 64
    for c in range(4):
        acc_s[pl.ds(c * 16, 16)] = zeros16
        acc_e[pl.ds(c * 16, 16)] = zeros16
    for r in range(16):
        pltpu.sync_copy(sh_s.at[r, pl.ds(col, 64)], tmp64)
        for c in range(4):
            sl = pl.ds(c * 16, 16)
            acc_s[sl] = acc_s[sl] + tmp64[sl]
        pltpu.sync_copy(sh_e.at[r, pl.ds(col, 64)], tmp64)
        for c in range(4):
            sl = pl.ds(c * 16, 16)
            acc_e[sl] = acc_e[sl] + tmp64[sl]
    for c in range(4):
        sl = pl.ds(c * 16, 16)
        acc_s[sl] = acc_e[sl] - acc_s[sl]
    pltpu.sync_copy(acc_s, p_hbm.at[pl.ds(col, 64)])


def _bincount(image_id):
    return pl.kernel(
        _bincount_body,
        out_type=jax.ShapeDtypeStruct((NUM_IMAGES,), jnp.int32),
        mesh=_sc_mesh(1),
        compiler_params=_SC_PARAMS,
        scratch_types=[
            pltpu.VMEM((_BCH + 16,), jnp.int32),
            pltpu.VMEM((16,), jnp.int32),
            pltpu.VMEM((NUM_IMAGES,), jnp.int32),
            pltpu.VMEM((NUM_IMAGES,), jnp.int32),
            pltpu.VMEM_SHARED((16, NUM_IMAGES), jnp.int32),
            pltpu.VMEM_SHARED((16, NUM_IMAGES), jnp.int32),
            pltpu.VMEM((64,), jnp.int32),
            pltpu.VMEM((64,), jnp.int32),
            pltpu.VMEM((64,), jnp.int32),
        ],
    )(image_id)


_BCH = N // 16


def _radix_body(g_hbm, ids_hbm, ltab_hbm, meta_hbm, cnts_hbm,
                gbuf, idbuf, ltab, ubuf, hist_loc, hsum, tmp4k, hscan,
                sc16a, sc16b, crow, hist_sh):
    wid = lax.axis_index("s")
    iota = _iota16()
    ones16 = jnp.ones((16,), jnp.int32)
    zeros16 = jnp.zeros((16,), jnp.int32)
    lane_off = iota * 2048

    def u32vec(x):
        return lax.broadcast(x, (16,)).astype(jnp.uint32)

    sc16a[pl.ds(16, 16)] = zeros16
    sc16b[pl.ds(16, 16)] = zeros16

    pltpu.sync_copy(ltab_hbm, ltab)
    half = _BCH // 2
    for h in range(2):
        pltpu.sync_copy(g_hbm.at[pl.ds(wid * _BCH + h * half, half)], gbuf)
        pltpu.sync_copy(ids_hbm.at[pl.ds(wid * _BCH + h * half, half)],
                        idbuf)

        def prep(i, _):
            for k in range(4):
                o = i * 64 + k * 16
                lt = plsc.load_gather(ltab, [idbuf[pl.ds(o, 16)]])
                s = lt + gbuf[pl.ds(o, 16)]
                bb = plsc.bitcast(s, jnp.uint32)
                hi = bb >= jnp.uint32(0x80000000)
                ubuf[pl.ds(h * half + o, 16)] = jnp.where(
                    hi, ~bb, bb | jnp.uint32(0x80000000))
            return 0
        lax.fori_loop(0, half // 64, prep, 0)

    def do_pass(nbins, bin_of, mask_of, target):
        nrows = nbins // 16

        def z(i, _):
            for k in range(8):
                hist_loc[pl.ds(i * 128 + k * 16, 16)] = zeros16
            return 0
        lax.fori_loop(0, 16 * 2048 // 128, z, 0)

        def vec(i, _):
            for k in range(4):
                u = ubuf[pl.ds(i * 64 + k * 16, 16)]
                idx = lane_off + bin_of(u)
                plsc.addupdate_scatter(hist_loc, [idx], ones16,
                                       mask=mask_of(u))
            return 0
        lax.fori_loop(0, _BCH // 64, vec, 0)

        def lr(c, _):
            acc = hist_loc[pl.ds(c * 16, 16)]
            for rr in range(1, 16):
                acc = acc + hist_loc[pl.ds(rr * 2048 + c * 16, 16)]
            hsum[pl.ds(c * 16, 16)] = acc
            return 0
        lax.fori_loop(0, nrows, lr, 0)
        pltpu.sync_copy(hsum.at[pl.ds(0, nbins)],
                        hist_sh.at[wid, pl.ds(0, nbins)])
        plsc.subcore_barrier()

        def zh(c, _):
            for k in range(4):
                hscan[pl.ds(c * 64 + k * 16, 16)] = zeros16
            return 0
        lax.fori_loop(0, nrows // 4, zh, 0)
        for rr in range(16):
            pltpu.sync_copy(hist_sh.at[rr, pl.ds(0, nbins)],
                            tmp4k.at[pl.ds(0, nbins)])

            def addrow(c, _):
                for k in range(4):
                    sl = pl.ds(c * 64 + k * 16, 16)
                    hscan[sl] = hscan[sl] + tmp4k[sl]
                return 0
            lax.fori_loop(0, nrows // 4, addrow, 0)
        plsc.subcore_barrier()

        def step(r2, carry):
            acc, found, b, above = carry
            r = nrows - 1 - r2
            row = hscan[pl.ds(r * 16, 16)]
            total = jnp.sum(row)
            crossed = jnp.logical_and(found == 0, acc + total >= target)

            def on_cross(_):
                rrev = lax.rev(row, (0,))
                suf = lax.rev(plsc.cumsum(rrev), (0,))
                cmask = ((acc + suf) >= target).astype(jnp.int32)
                ffs = plsc.all_reduce_ffs(lax.rev(cmask, (0,)) != 0)
                f0 = ffs[0] if getattr(ffs, "ndim", 0) else ffs
                lstar = 15 - f0
                sc16a[pl.ds(0, 16)] = suf
                sc16b[pl.ds(0, 16)] = row
                sufl = sc16a[pl.ds(lstar, 16)][0]
                rowl = sc16b[pl.ds(lstar, 16)][0]
                return r * 16 + lstar, acc + sufl - rowl

            def no_cross(_):
                return b, above

            b_n, ab_n = lax.cond(crossed, on_cross, no_cross, 0)
            return (acc + total, found | crossed.astype(jnp.int32),
                    b_n, ab_n)

        _, _, b, above = lax.fori_loop(
            0, nrows, step,
            (jnp.int32(0), jnp.int32(0), jnp.int32(0), jnp.int32(0)))
        return b, above

    b1, ab1 = do_pass(
        2048,
        lambda u: (u >> 21).astype(jnp.int32),
        lambda u: None,
        jnp.int32(K))
    k2 = K - ab1
    b2, ab2 = do_pass(
        2048,
        lambda u: ((u >> 10) & jnp.uint32(0x7FF)).astype(jnp.int32),
        lambda u: (u >> 21) == u32vec(b1),
        k2)
    k3 = k2 - ab2
    b3, ab3 = do_pass(
        1024,
        lambda u: (u & jnp.uint32(0x3FF)).astype(jnp.int32),
        lambda u: (u >> 10) == u32vec((b1 << 11) | b2),
        k3)
    r = k3 - ab3
    t_i32 = (b1 << 21) | (b2 << 10) | b3
    tvec = plsc.bitcast(lax.broadcast(t_i32, (16,)), jnp.uint32)

    def count_half(h):
        def vec(i, c2):
            g, ti = c2
            u = ubuf[pl.ds(h * (_BCH // 2) + i * 16, 16)]
            g = g + plsc.all_reduce_population_count(u > tvec)
            ti = ti + plsc.all_reduce_population_count(u == tvec)
            return g, ti
        return lax.fori_loop(0, _BCH // 2 // 16, vec, (zeros16, zeros16))

    g0, t0 = count_half(0)
    g1, t1 = count_half(1)
    zf = zeros16
    vout = (jnp.where(iota == 0, g0, zf) + jnp.where(iota == 1, t0, zf)
            + jnp.where(iota == 8, g1, zf) + jnp.where(iota == 9, t1, zf))
    crow[pl.ds(0, 16)] = vout
    pltpu.sync_copy(crow, cnts_hbm.at[pl.ds(wid * 16, 16)])

    @pl.when(wid == 0)
    def _():
        tv = lax.broadcast(t_i32, (16,))
        rv = lax.broadcast(r, (16,))
        mv = jnp.where(iota == 0, tv, jnp.where(iota == 1, rv, zf))
        crow[pl.ds(0, 16)] = mv
        pltpu.sync_copy(crow, meta_hbm)


def _threshold(g, image_id, logtab):
    return pl.kernel(
        _radix_body,
        out_type=(jax.ShapeDtypeStruct((16,), jnp.int32),
                  jax.ShapeDtypeStruct((_NW * 8,), jnp.int32)),
        mesh=_sc_mesh(1),
        compiler_params=_SC_PARAMS,
        scratch_types=[
            pltpu.VMEM((_BCH // 2,), jnp.float32),
            pltpu.VMEM((_BCH // 2,), jnp.int32),
            pltpu.VMEM((NUM_IMAGES,), jnp.float32),
            pltpu.VMEM((_BCH,), jnp.uint32),
            pltpu.VMEM((16 * 2048,), jnp.int32),
            pltpu.VMEM((2048,), jnp.int32),
            pltpu.VMEM((2048,), jnp.int32),
            pltpu.VMEM((2048,), jnp.int32),
            pltpu.VMEM((32,), jnp.int32),
            pltpu.VMEM((32,), jnp.int32),
            pltpu.VMEM((16,), jnp.int32),
            pltpu.VMEM_SHARED((16, 2048), jnp.int32),
        ],
    )(g, image_id, logtab)


_NW = 32
_CHUNK_C = N // _NW
_SUB_C = 2048
_LOC = 16640


def _compact_body(g_hbm, ids_hbm, ltab_hbm, data_hbm, meta_hbm, cnts_hbm,
                  x_hbm, selid_hbm,
                  mbuf, cbuf, sbuf, ibuf, ltab, idxloc, idloc,
                  rows128, rows8):
    wid = lax.axis_index("s") * 2 + lax.axis_index("c")
    iota = _iota16()

    pltpu.sync_copy(meta_hbm, mbuf)
    pltpu.sync_copy(cnts_hbm, cbuf.at[pl.ds(0, _NW * 8)])
    mvec = mbuf[pl.ds(0, 16)]
    t_i32 = mvec[0]
    r = mvec[1]
    tvec = plsc.bitcast(lax.broadcast(t_i32, (16,)), jnp.uint32)

    def pref(w2, carry):
        off, tiepre, my_n, my_a = carry
        cv = cbuf[pl.ds(w2 * 8, 16)]
        gt = cv[0]
        tie = cv[1]
        a = jnp.clip(r - tiepre, 0, tie)
        nw = gt + a
        mw = (nw + 7) & (-8)
        off = jnp.where(w2 < wid, off + mw, off)
        my_n = jnp.where(w2 == wid, nw, my_n)
        my_a = jnp.where(w2 == wid, a, my_a)
        tiepre = jnp.where(w2 < wid, tiepre + tie, tiepre)
        return off, tiepre, my_n, my_a
    off, _, n_w, a_w = lax.fori_loop(
        0, _NW, pref, (jnp.int32(0), jnp.int32(0), jnp.int32(0),
                       jnp.int32(0)))
    m_w = jnp.where(wid == _NW - 1, M_BUF - off, (n_w + 7) & (-8))
    off = pl.multiple_of(off, 8)
    m_w = pl.multiple_of(m_w, 8)

    chunk_base = wid * _CHUNK_C
    zero_splat = jnp.zeros((16,), jnp.int32)

    pltpu.sync_copy(g_hbm.at[pl.ds(chunk_base, _CHUNK_C)], sbuf)
    pltpu.sync_copy(ids_hbm.at[pl.ds(chunk_base, _CHUNK_C)], ibuf)
    pltpu.sync_copy(ltab_hbm, ltab)

    def monotone(o):
        lt = plsc.load_gather(ltab, [ibuf[pl.ds(o, 16)]])
        s = lt + sbuf[pl.ds(o, 16)]
        bb = plsc.bitcast(s, jnp.uint32)
        hi = bb >= jnp.uint32(0x80000000)
        return jnp.where(hi, ~bb, bb | jnp.uint32(0x80000000))

    def emit(o, pos, mask):
        idxvec = chunk_base + o + iota
        plsc.store_scatter(idxloc, [pos], idxvec, mask=mask)
        idv = ibuf[pl.ds(o, 16)]
        plsc.store_scatter(idloc, [pos], idv, mask=mask)

    def vec_ties(i, vc):
        ptr, tiec = vc
        for k in range(2):
            o = i * 32 + k * 16
            u = monotone(o)
            mgt = u > tvec
            mtie = u == tvec
            cumt = plsc.cumsum(mtie.astype(jnp.int32))
            rank = tiec + cumt - 1
            macc = mtie & (rank < a_w)
            mask = mgt | macc
            cums = plsc.cumsum(mask.astype(jnp.int32))
            pos = ptr + cums - 1
            emit(o, pos, mask)
            ptr = ptr + plsc.all_reduce_population_count(mask)
            tiec = tiec + plsc.all_reduce_population_count(mtie)
        return ptr, tiec

    def vec_plain(i, ptr):
        for k in range(2):
            o = i * 32 + k * 16
            u = monotone(o)
            mask = u > tvec
            cums = plsc.cumsum(mask.astype(jnp.int32))
            pos = ptr + cums - 1
            emit(o, pos, mask)
            ptr = ptr + plsc.all_reduce_population_count(mask)
        return ptr

    def scan_ties(_):
        return lax.fori_loop(0, _CHUNK_C // 32, vec_ties,
                             (zero_splat, zero_splat))[0]

    def scan_plain(_):
        return lax.fori_loop(0, _CHUNK_C // 32, vec_plain, zero_splat)

    lax.cond(a_w > 0, scan_ties, scan_plain, 0)

    sent = jnp.full((16,), SENTINEL_ID, jnp.int32)

    def fill(j, _):
        pos = n_w + j * 16 + iota
        mask = pos < m_w
        plsc.store_scatter(idxloc, [pos], zero_splat, mask=mask)
        plsc.store_scatter(idloc, [pos], sent, mask=mask)
        return 0
    lax.fori_loop(0, (m_w - n_w + 15) // 16, fill, 0)

    nfull = m_w // 128

    def gchunk(c, _):
        lo = pl.multiple_of(c * 128, 128)
        dst = pl.multiple_of(off + lo, 8)
        pltpu.sync_copy(data_hbm.at[idxloc.at[pl.ds(lo, 128)]], rows128)
        pltpu.sync_copy(rows128, x_hbm.at[pl.ds(dst, 128)])
        pltpu.sync_copy(idloc.at[pl.ds(lo, 128)],
                        selid_hbm.at[pl.ds(dst, 128)])
        return 0
    lax.fori_loop(0, nfull, gchunk, 0)

    ntail = (m_w - nfull * 128) // 8

    def gtail(c, _):
        lo = pl.multiple_of(nfull * 128 + c * 8, 8)
        dst = pl.multiple_of(off + lo, 8)
        pltpu.sync_copy(data_hbm.at[idxloc.at[pl.ds(lo, 8)]], rows8)
        pltpu.sync_copy(rows8, x_hbm.at[pl.ds(dst, 8)])
        pltpu.sync_copy(idloc.at[pl.ds(lo, 8)],
                        selid_hbm.at[pl.ds(dst, 8)])
        return 0
    lax.fori_loop(0, ntail, gtail, 0)


def _compact_gather(data, image_id, g, logtab, meta, cnts):
    return pl.kernel(
        _compact_body,
        out_type=(jax.ShapeDtypeStruct((M_BUF, WIDTH), jnp.float32),
                  jax.ShapeDtypeStruct((M_BUF,), jnp.int32)),
        mesh=_sc_mesh(2),
        compiler_params=_SC_PARAMS,
        scratch_types=[
            pltpu.VMEM((16,), jnp.int32),
            pltpu.VMEM((_NW * 8 + 16,), jnp.int32),
            pltpu.VMEM((_CHUNK_C,), jnp.float32),
            pltpu.VMEM((_CHUNK_C,), jnp.int32),
            pltpu.VMEM((NUM_IMAGES,), jnp.float32),
            pltpu.VMEM((_LOC,), jnp.int32),
            pltpu.VMEM((_LOC,), jnp.int32),
            pltpu.VMEM((128, WIDTH), jnp.float32),
            pltpu.VMEM((8, WIDTH), jnp.float32),
        ],
    )(g, image_id, logtab, data, meta, cnts)


def _mlp_kernel(x_ref, w0_ref, b0_ref, w1_ref, b1_ref, o_ref):
    h = jnp.dot(x_ref[...], w0_ref[...], preferred_element_type=jnp.float32)
    h = jnp.maximum(h + b0_ref[...], 0.0)
    o = jnp.dot(h, w1_ref[...], preferred_element_type=jnp.float32)
    o_ref[...] = o + b1_ref[...]


def _mlp(x, W0, b0, W1s, b1s):
    rows = 256
    grid = (M_BUF // rows,)
    return pl.pallas_call(
        _mlp_kernel,
        grid=grid,
        in_specs=[
            pl.BlockSpec((rows, WIDTH), lambda i: (i, 0)),
            pl.BlockSpec((WIDTH, WIDTH), lambda i: (0, 0)),
            pl.BlockSpec((1, WIDTH), lambda i: (0, 0)),
            pl.BlockSpec((WIDTH, WIDTH), lambda i: (0, 0)),
            pl.BlockSpec((1, WIDTH), lambda i: (0, 0)),
        ],
        out_specs=pl.BlockSpec((rows, WIDTH), lambda i: (i, 0)),
        out_shape=jax.ShapeDtypeStruct((M_BUF, WIDTH), jnp.float32),
    )(x, W0, b0.reshape(1, WIDTH), W1s, b1s.reshape(1, WIDTH))


_E_ROWS = M_BUF // 16
_E_FULL = _E_ROWS // 128
_E_TAIL = _E_ROWS - _E_FULL * 128
_ACC_ROWS = 1040


def _segsum_body(rows_hbm, ids_hbm, out_hbm,
                 zbuf, ids_a, rows_a, ids_b, rows_b, ids16, rows16, obuf,
                 sem_a, sem_b, acc_sh):
    wid = lax.axis_index("s")
    z16 = jnp.zeros((16,), jnp.float32)

    def zrow(r, _):
        for c in range(WIDTH // 16):
            zbuf[r, pl.ds(c * 16, 16)] = z16
        return 0
    lax.fori_loop(0, _ACC_ROWS // 16, zrow, 0)
    pltpu.sync_copy(zbuf, acc_sh.at[pl.ds(wid * (_ACC_ROWS // 16),
                                          _ACC_ROWS // 16)])
    plsc.subcore_barrier()

    base = wid * _E_ROWS
    bufs = [(ids_a, rows_a, sem_a), (ids_b, rows_b, sem_b)]

    def issue(c):
        ids_r, rows_r, sem = bufs[c % 2]
        off = base + c * 128
        d1 = pltpu.async_copy(ids_hbm.at[pl.ds(off, 128)], ids_r, sem)
        d2 = pltpu.async_copy(rows_hbm.at[pl.ds(off, 128)], rows_r, sem)
        return d1, d2

    pend = issue(0)
    for c in range(_E_FULL):
        pend[0].wait()
        pend[1].wait()
        if c + 1 < _E_FULL:
            nxt = issue(c + 1)
        ids_r, rows_r, _ = bufs[c % 2]
        pltpu.sync_copy(rows_r, acc_sh.at[ids_r], add=True)
        if c + 1 < _E_FULL:
            pend = nxt
    toff = base + _E_FULL * 128
    pltpu.sync_copy(ids_hbm.at[pl.ds(toff, _E_TAIL)], ids16)
    pltpu.sync_copy(rows_hbm.at[pl.ds(toff, _E_TAIL)], rows16)
    pltpu.sync_copy(rows16, acc_sh.at[ids16], add=True)
    plsc.subcore_barrier()

    pltpu.sync_copy(acc_sh.at[pl.ds(wid * 64, 64)], obuf)
    pltpu.sync_copy(obuf, out_hbm.at[pl.ds(wid * 64, 64)])


def _segsum(out_rows, sel_ids):
    return pl.kernel(
        _segsum_body,
        out_type=jax.ShapeDtypeStruct((NUM_IMAGES, WIDTH), jnp.float32),
        mesh=_sc_mesh(1),
        compiler_params=_SC_PARAMS,
        scratch_types=[
            pltpu.VMEM((_ACC_ROWS // 16, WIDTH), jnp.float32),
            pltpu.VMEM((128,), jnp.int32),
            pltpu.VMEM((128, WIDTH), jnp.float32),
            pltpu.VMEM((128,), jnp.int32),
            pltpu.VMEM((128, WIDTH), jnp.float32),
            pltpu.VMEM((16,), jnp.int32),
            pltpu.VMEM((16, WIDTH), jnp.float32),
            pltpu.VMEM((64, WIDTH), jnp.float32),
            pltpu.SemaphoreType.DMA,
            pltpu.SemaphoreType.DMA,
            pltpu.VMEM_SHARED((_ACC_ROWS, WIDTH), jnp.float32),
        ],
    )(out_rows, sel_ids)


def kernel(data, image_id, sample_size, W0, b0, W1, b1):
    counts = _bincount(image_id)
    logtab = jnp.log((jnp.float32(1.0) / counts.astype(jnp.float32))
                     / jnp.float32(NUM_IMAGES))
    g = jax.random.gumbel(jax.random.key(1234), (N,), dtype=jnp.float32)

    meta, cnts = _threshold(g, image_id, logtab)
    x, ids = _compact_gather(data, image_id, g, logtab, meta, cnts)

    inv = 1.0 / jnp.asarray(sample_size).astype(jnp.float32)
    out_rows = _mlp(x, W0, b0, W1 * inv, b1 * inv)
    return _segsum(out_rows, ids)
